# Initial kernel scaffold; baseline (speedup 1.0000x reference)
import jax
import jax.numpy as jnp
from jax.experimental import pallas as pl


def kernel(x, edge_index, node_type, edge_type, k_w_comp, k_weight, q_w_comp, q_weight, v_w_comp, v_weight, a_w_comp, a_weight, relation_pri, relation_att, relation_msg, loop_weight, bias):
    raise NotImplementedError("write your pallas kernel here")


# speedup vs baseline: 27.2267x; 27.2267x over previous
import functools

import jax
import jax.numpy as jnp
from jax import lax
from jax.experimental import pallas as pl
from jax.experimental.pallas import tpu as pltpu
from jax.experimental.pallas import tpu_sc as plsc

N = 10000
E = 320000
D = 128
NT = 3
ET = 5
H = 4
DK = 32
NB = 2
SCALE = float((D / H) ** (-0.5))

NC = 2
NS = 16
NW = NC * NS
CH = 128
NCHUNK_PAD = ((E // CH) + NW - 1) // NW * NW
CPW = NCHUNK_PAD // NW
EPAD = NCHUNK_PAD * CH
WW = 144
NP = 10240

BN = 1000
BE = 2048
NBLK_E = EPAD // BE


def _a_body(x_ref, nt_ref, kw_ref, kc_ref, qw_ref, qc_ref, vw_ref, vc_ref,
            aw_att_ref, aw_msg_ref, pri_ref, kv_ref, q_ref):
    x = x_ref[...]
    nt = nt_ref[...]

    def _proj(w_ref, c_ref):
        y0 = jnp.dot(x, w_ref[0], preferred_element_type=jnp.float32)
        y1 = jnp.dot(x, w_ref[1], preferred_element_type=jnp.float32)
        c0 = sum(jnp.where(nt == t, c_ref[t, 0], 0.0) for t in range(NT))
        c1 = sum(jnp.where(nt == t, c_ref[t, 1], 0.0) for t in range(NT))
        return c0 * y0 + c1 * y1

    k = _proj(kw_ref, kc_ref)
    v = _proj(vw_ref, vc_ref)
    q_ref[...] = _proj(qw_ref, qc_ref)

    for t in range(ET):
        cols = []
        for h in range(H):
            s = pri_ref[t, h] * SCALE
            cols.append(jnp.dot(k[:, h * DK:(h + 1) * DK], aw_att_ref[t, h],
                                preferred_element_type=jnp.float32) * s)
        for h in range(H):
            cols.append(jnp.dot(v[:, h * DK:(h + 1) * DK], aw_msg_ref[t, h],
                                preferred_element_type=jnp.float32))
        kv_ref[t] = jnp.concatenate(cols, axis=1)


def _dense_prep(x, node_type, kw, kc, qw, qc, vw, vc, att_w, msg_w, pri):
    full = lambda shp: pl.BlockSpec(shp, lambda i: tuple(0 for _ in shp))
    return pl.pallas_call(
        _a_body,
        grid=(N // BN,),
        in_specs=[
            pl.BlockSpec((BN, D), lambda i: (i, 0)),
            pl.BlockSpec((BN, 1), lambda i: (i, 0)),
            full((NB, D, D)), full((NT, NB)),
            full((NB, D, D)), full((NT, NB)),
            full((NB, D, D)), full((NT, NB)),
            full((ET, H, DK, DK)), full((ET, H, DK, DK)),
            full((ET, H)),
        ],
        out_specs=[
            pl.BlockSpec((ET, BN, 2 * D), lambda i: (0, i, 0)),
            pl.BlockSpec((BN, D), lambda i: (i, 0)),
        ],
        out_shape=[
            jax.ShapeDtypeStruct((ET, N, 2 * D), jnp.float32),
            jax.ShapeDtypeStruct((N, D), jnp.float32),
        ],
    )(x, node_type, kw, kc, qw, qc, vw, vc, att_w, msg_w, pri)


def _gather_call(kv, q, gidx, didx):
    mesh = plsc.VectorSubcoreMesh(core_axis_name="c", subcore_axis_name="s")

    @functools.partial(
        pl.kernel, mesh=mesh,
        out_type=[
            jax.ShapeDtypeStruct((EPAD, 2 * D), jnp.float32),
            jax.ShapeDtypeStruct((EPAD, D), jnp.float32),
        ],
        scratch_types=[
            pltpu.VMEM((CH,), jnp.int32),
            pltpu.VMEM((CH,), jnp.int32),
            pltpu.VMEM((CH, 2 * D), jnp.float32),
            pltpu.VMEM((CH, D), jnp.float32),
        ],
    )
    def _gather(kv_hbm, q_hbm, gidx_hbm, didx_hbm, okv_hbm, oq_hbm,
                idx_v, didx_v, kvbuf, qbuf):
        wid = lax.axis_index("c") * NS + lax.axis_index("s")

        @pl.loop(0, CPW)
        def _(j):
            pltpu.sync_copy(gidx_hbm.at[wid, j], idx_v)
            pltpu.sync_copy(didx_hbm.at[wid, j], didx_v)
            pltpu.sync_copy(kv_hbm.at[idx_v], kvbuf)
            pltpu.sync_copy(q_hbm.at[didx_v], qbuf)
            g = (wid * CPW + j) * CH
            pltpu.sync_copy(kvbuf, okv_hbm.at[pl.ds(g, CH)])
            pltpu.sync_copy(qbuf, oq_hbm.at[pl.ds(g, CH)])

    return _gather(kv, q, gidx, didx)


def _c_body(kv_ref, q_ref, w_ref):
    kvg = kv_ref[...]
    krel = kvg[:, :D]
    vrel = kvg[:, D:]
    prod = krel * q_ref[...]
    chunks = []
    exps = []
    for h in range(H):
        a = jnp.sum(prod[:, h * DK:(h + 1) * DK], axis=1, keepdims=True)
        e = jnp.exp(a)
        exps.append(e)
        chunks.append(vrel[:, h * DK:(h + 1) * DK] * e)
    tail = jnp.concatenate(exps + [jnp.zeros((BE, WW - D - H), jnp.float32)],
                           axis=1)
    w_ref[...] = jnp.concatenate(chunks + [tail], axis=1)


def _edge_math(kvg, qg):
    return pl.pallas_call(
        _c_body,
        grid=(NBLK_E,),
        in_specs=[
            pl.BlockSpec((BE, 2 * D), lambda i: (i, 0)),
            pl.BlockSpec((BE, D), lambda i: (i, 0)),
        ],
        out_specs=pl.BlockSpec((BE, WW), lambda i: (i, 0)),
        out_shape=jax.ShapeDtypeStruct((EPAD, WW), jnp.float32),
    )(kvg, qg)


def _scatter_call(w, sdst, zrow):
    mesh = plsc.VectorSubcoreMesh(core_axis_name="c", subcore_axis_name="s")
    rps = NP // NS
    zc = rps // CH

    @functools.partial(
        pl.kernel, mesh=mesh,
        out_type=jax.ShapeDtypeStruct((NC, NP, WW), jnp.float32),
        compiler_params=pltpu.CompilerParams(use_tc_tiling_on_sc=False),
        scratch_types=[
            pltpu.VMEM((CH,), jnp.int32),
            pltpu.VMEM((CH, WW), jnp.float32),
            pltpu.VMEM_SHARED((NP, WW), jnp.float32),
        ],
    )
    def _scatter(w_hbm, sdst_hbm, z_hbm, out_hbm, idx_v, wbuf, acc):
        cid = lax.axis_index("c")
        sid = lax.axis_index("s")
        wid = cid * NS + sid

        @pl.loop(0, zc)
        def _(z):
            pltpu.sync_copy(z_hbm, acc.at[pl.ds(sid * rps + z * CH, CH)])

        plsc.subcore_barrier()

        @pl.loop(0, CPW)
        def _(j):
            pltpu.sync_copy(sdst_hbm.at[wid, j], idx_v)
            pltpu.sync_copy(w_hbm.at[pl.ds((wid * CPW + j) * CH, CH)], wbuf)
            pltpu.sync_copy(wbuf, acc.at[idx_v], add=True)

        plsc.subcore_barrier()

        @pl.loop(0, zc)
        def _(z):
            r = sid * rps + z * CH
            pltpu.sync_copy(acc.at[pl.ds(r, CH)], out_hbm.at[cid, pl.ds(r, CH)])

    return _scatter(w, sdst, zrow)


def _e_body(p_ref, x_ref, nt_ref, aw_ref, ac_ref, lw_ref, b_ref, o_ref):
    tot = p_ref[0] + p_ref[1]
    numer = tot[:, :D]
    cols = []
    for h in range(H):
        d = tot[:, D + h:D + h + 1]
        cols.append(numer[:, h * DK:(h + 1) * DK] / (d + 1e-9))
    agg = jnp.concatenate(cols, axis=1)
    y0 = jnp.dot(agg, aw_ref[0], preferred_element_type=jnp.float32)
    y1 = jnp.dot(agg, aw_ref[1], preferred_element_type=jnp.float32)
    nt = nt_ref[...]
    c0 = sum(jnp.where(nt == t, ac_ref[t, 0], 0.0) for t in range(NT))
    c1 = sum(jnp.where(nt == t, ac_ref[t, 1], 0.0) for t in range(NT))
    h_out = c0 * y0 + c1 * y1
    h_out = h_out + jnp.dot(x_ref[...], lw_ref[...],
                            preferred_element_type=jnp.float32)
    h_out = h_out + b_ref[...]
    o_ref[...] = jnp.maximum(h_out, 0.0)


def _finalize(partials, x, node_type, aw, ac, lw, bias):
    full = lambda shp: pl.BlockSpec(shp, lambda i: tuple(0 for _ in shp))
    return pl.pallas_call(
        _e_body,
        grid=(N // BN,),
        in_specs=[
            pl.BlockSpec((NC, BN, WW), lambda i: (0, i, 0)),
            pl.BlockSpec((BN, D), lambda i: (i, 0)),
            pl.BlockSpec((BN, 1), lambda i: (i, 0)),
            full((NB, D, D)), full((NT, NB)), full((D, D)),
            pl.BlockSpec((1, D), lambda i: (0, 0)),
        ],
        out_specs=pl.BlockSpec((BN, D), lambda i: (i, 0)),
        out_shape=jax.ShapeDtypeStruct((N, D), jnp.float32),
    )(partials, x, node_type, aw, ac, lw, bias)


def kernel(x, edge_index, node_type, edge_type, k_w_comp, k_weight,
           q_w_comp, q_weight, v_w_comp, v_weight, a_w_comp, a_weight,
           relation_pri, relation_att, relation_msg, loop_weight, bias):
    src = edge_index[0]
    dst = edge_index[1]
    nt2 = node_type.reshape(N, 1)
    kw = k_weight.reshape(NB, D, D)
    qw = q_weight.reshape(NB, D, D)
    vw = v_weight.reshape(NB, D, D)
    aw = a_weight.reshape(NB, D, D)

    kv, q = _dense_prep(x, nt2, kw, k_w_comp, qw, q_w_comp, vw, v_w_comp,
                        relation_att, relation_msg, relation_pri)
    kv2 = kv.reshape(ET * N, 2 * D)

    pad = EPAD - E
    gidx = jnp.concatenate([edge_type * N + src,
                            jnp.zeros((pad,), jnp.int32)]).reshape(NW, CPW, CH)
    didx = jnp.concatenate([dst, jnp.zeros((pad,), jnp.int32)]
                           ).reshape(NW, CPW, CH)
    sdst = jnp.concatenate([dst, jnp.full((pad,), N, jnp.int32)]
                           ).reshape(NW, CPW, CH)

    kvg, qg = _gather_call(kv2, q, gidx, didx)
    w = _edge_math(kvg, qg)
    partials = _scatter_call(w, sdst, jnp.zeros((CH, WW), jnp.float32))
    return _finalize(partials, x, nt2, aw, a_w_comp, loop_weight,
                     bias.reshape(1, D))


# speedup vs baseline: 31.2438x; 1.1475x over previous
import functools

import jax
import jax.numpy as jnp
from jax import lax
from jax.experimental import pallas as pl
from jax.experimental.pallas import tpu as pltpu
from jax.experimental.pallas import tpu_sc as plsc

N = 10000
E = 320000
D = 128
NT = 3
ET = 5
H = 4
DK = 32
NB = 2
SCALE = float((D / H) ** (-0.5))

NC = 2
NS = 16
NW = NC * NS
CH = 128
CPW = 80
NCHUNK_PAD = CPW * NW
EPAD = NCHUNK_PAD * CH
WW = 144
NP = 10240

BN = 1000
BE = 2048
NBLK_E = EPAD // BE


def _a_body(x_ref, nt_ref, kw_ref, kc_ref, qw_ref, qc_ref, vw_ref, vc_ref,
            aw_att_ref, aw_msg_ref, pri_ref, kv_ref, q_ref):
    x = x_ref[...]
    nt = nt_ref[...]

    def _proj(w_ref, c_ref):
        y0 = jnp.dot(x, w_ref[0], preferred_element_type=jnp.float32)
        y1 = jnp.dot(x, w_ref[1], preferred_element_type=jnp.float32)
        c0 = sum(jnp.where(nt == t, c_ref[t, 0], 0.0) for t in range(NT))
        c1 = sum(jnp.where(nt == t, c_ref[t, 1], 0.0) for t in range(NT))
        return c0 * y0 + c1 * y1

    k = _proj(kw_ref, kc_ref)
    v = _proj(vw_ref, vc_ref)
    q_ref[...] = _proj(qw_ref, qc_ref)

    for t in range(ET):
        cols = []
        for h in range(H):
            s = pri_ref[t, h] * SCALE
            cols.append(jnp.dot(k[:, h * DK:(h + 1) * DK], aw_att_ref[t, h],
                                preferred_element_type=jnp.float32) * s)
        for h in range(H):
            cols.append(jnp.dot(v[:, h * DK:(h + 1) * DK], aw_msg_ref[t, h],
                                preferred_element_type=jnp.float32))
        kv_ref[t] = jnp.concatenate(cols, axis=1)


def _dense_prep(x, node_type, kw, kc, qw, qc, vw, vc, att_w, msg_w, pri):
    full = lambda shp: pl.BlockSpec(shp, lambda i: tuple(0 for _ in shp))
    return pl.pallas_call(
        _a_body,
        grid=(N // BN,),
        in_specs=[
            pl.BlockSpec((BN, D), lambda i: (i, 0)),
            pl.BlockSpec((BN, 1), lambda i: (i, 0)),
            full((NB, D, D)), full((NT, NB)),
            full((NB, D, D)), full((NT, NB)),
            full((NB, D, D)), full((NT, NB)),
            full((ET, H, DK, DK)), full((ET, H, DK, DK)),
            full((ET, H)),
        ],
        out_specs=[
            pl.BlockSpec((ET, BN, 2 * D), lambda i: (0, i, 0)),
            pl.BlockSpec((BN, D), lambda i: (i, 0)),
        ],
        out_shape=[
            jax.ShapeDtypeStruct((ET, N, 2 * D), jnp.float32),
            jax.ShapeDtypeStruct((N, D), jnp.float32),
        ],
    )(x, node_type, kw, kc, qw, qc, vw, vc, att_w, msg_w, pri)


def _gather_call(kv, q, gidx, didx):
    mesh = plsc.VectorSubcoreMesh(core_axis_name="c", subcore_axis_name="s")

    @functools.partial(
        pl.kernel, mesh=mesh,
        out_type=[
            jax.ShapeDtypeStruct((EPAD, 2 * D), jnp.float32),
            jax.ShapeDtypeStruct((EPAD, D), jnp.float32),
        ],
        scratch_types=[
            pltpu.VMEM((CPW, CH), jnp.int32),
            pltpu.VMEM((CPW, CH), jnp.int32),
            pltpu.VMEM((2, CH, 2 * D), jnp.float32),
            pltpu.VMEM((2, CH, D), jnp.float32),
            [pltpu.SemaphoreType.DMA] * 8,
        ],
    )
    def _gather(kv_hbm, q_hbm, gidx_hbm, didx_hbm, okv_hbm, oq_hbm,
                gidx_v, didx_v, kvbuf, qbuf, sems):
        wid = lax.axis_index("c") * NS + lax.axis_index("s")
        sg_kv, sg_q, so_kv, so_q = sems[0:2], sems[2:4], sems[4:6], sems[6:8]
        base = wid * CPW * CH

        pltpu.sync_copy(gidx_hbm.at[wid], gidx_v)
        pltpu.sync_copy(didx_hbm.at[wid], didx_v)

        def start_gather(j, b):
            pltpu.make_async_copy(kv_hbm.at[gidx_v.at[j]], kvbuf.at[b],
                                  sg_kv[b]).start()
            pltpu.make_async_copy(q_hbm.at[didx_v.at[j]], qbuf.at[b],
                                  sg_q[b]).start()

        def wait_gather(j, b):
            pltpu.make_async_copy(kv_hbm.at[gidx_v.at[j]], kvbuf.at[b],
                                  sg_kv[b]).wait()
            pltpu.make_async_copy(q_hbm.at[didx_v.at[j]], qbuf.at[b],
                                  sg_q[b]).wait()

        def start_out(j, b):
            g = base + j * CH
            pltpu.make_async_copy(kvbuf.at[b], okv_hbm.at[pl.ds(g, CH)],
                                  so_kv[b]).start()
            pltpu.make_async_copy(qbuf.at[b], oq_hbm.at[pl.ds(g, CH)],
                                  so_q[b]).start()

        def wait_out(b):
            pltpu.make_async_copy(kvbuf.at[b], okv_hbm.at[pl.ds(0, CH)],
                                  so_kv[b]).wait()
            pltpu.make_async_copy(qbuf.at[b], oq_hbm.at[pl.ds(0, CH)],
                                  so_q[b]).wait()

        start_gather(0, 0)

        @pl.loop(0, CPW // 2)
        def _(jj):
            j0 = jj * 2
            @pl.when(jj > 0)
            def _():
                wait_out(1)
            start_gather(j0 + 1, 1)
            wait_gather(j0, 0)
            start_out(j0, 0)
            wait_out(0)

            @pl.when(jj < CPW // 2 - 1)
            def _():
                start_gather(j0 + 2, 0)
            wait_gather(j0 + 1, 1)
            start_out(j0 + 1, 1)

        wait_out(1)

    return _gather(kv, q, gidx, didx)


def _c_body(kv_ref, q_ref, w_ref):
    kvg = kv_ref[...]
    krel = kvg[:, :D]
    vrel = kvg[:, D:]
    prod = krel * q_ref[...]
    chunks = []
    exps = []
    for h in range(H):
        a = jnp.sum(prod[:, h * DK:(h + 1) * DK], axis=1, keepdims=True)
        e = jnp.exp(a)
        exps.append(e)
        chunks.append(vrel[:, h * DK:(h + 1) * DK] * e)
    tail = jnp.concatenate(exps + [jnp.zeros((BE, WW - D - H), jnp.float32)],
                           axis=1)
    w_ref[...] = jnp.concatenate(chunks + [tail], axis=1)


def _edge_math(kvg, qg):
    return pl.pallas_call(
        _c_body,
        grid=(NBLK_E,),
        in_specs=[
            pl.BlockSpec((BE, 2 * D), lambda i: (i, 0)),
            pl.BlockSpec((BE, D), lambda i: (i, 0)),
        ],
        out_specs=pl.BlockSpec((BE, WW), lambda i: (i, 0)),
        out_shape=jax.ShapeDtypeStruct((EPAD, WW), jnp.float32),
    )(kvg, qg)


def _scatter_call(w, sdst, zrow):
    mesh = plsc.VectorSubcoreMesh(core_axis_name="c", subcore_axis_name="s")
    rps = NP // NS
    zc = rps // CH

    @functools.partial(
        pl.kernel, mesh=mesh,
        out_type=jax.ShapeDtypeStruct((NC, NP, WW), jnp.float32),
        compiler_params=pltpu.CompilerParams(use_tc_tiling_on_sc=False),
        scratch_types=[
            pltpu.VMEM((2, CH), jnp.int32),
            pltpu.VMEM((2, CH, WW), jnp.float32),
            pltpu.VMEM_SHARED((NP, WW), jnp.float32),
            [pltpu.SemaphoreType.DMA] * 4,
        ],
    )
    def _scatter(w_hbm, sdst_hbm, z_hbm, out_hbm, ibuf, wbuf, acc, sems):
        cid = lax.axis_index("c")
        sid = lax.axis_index("s")
        wid = cid * NS + sid
        base = wid * CPW * CH
        sw, si = sems[0:2], sems[2:4]

        @pl.loop(0, zc)
        def _(z):
            pltpu.sync_copy(z_hbm, acc.at[pl.ds(sid * rps + z * CH, CH)])

        plsc.subcore_barrier()

        def start_ld(j, b):
            pltpu.make_async_copy(w_hbm.at[pl.ds(base + j * CH, CH)],
                                  wbuf.at[b], sw[b]).start()
            pltpu.make_async_copy(sdst_hbm.at[wid, j], ibuf.at[b],
                                  si[b]).start()

        def wait_ld(j, b):
            pltpu.make_async_copy(w_hbm.at[pl.ds(base + j * CH, CH)],
                                  wbuf.at[b], sw[b]).wait()
            pltpu.make_async_copy(sdst_hbm.at[wid, j], ibuf.at[b],
                                  si[b]).wait()

        def scat(j, b):
            pltpu.sync_copy(wbuf.at[b], acc.at[ibuf.at[b]], add=True)

        start_ld(0, 0)

        @pl.loop(0, CPW // 2)
        def _(jj):
            j0 = jj * 2
            start_ld(j0 + 1, 1)
            wait_ld(j0, 0)
            scat(j0, 0)

            @pl.when(jj < CPW // 2 - 1)
            def _():
                start_ld(j0 + 2, 0)
            wait_ld(j0 + 1, 1)
            scat(j0 + 1, 1)

        plsc.subcore_barrier()

        @pl.loop(0, zc)
        def _(z):
            r = sid * rps + z * CH
            pltpu.sync_copy(acc.at[pl.ds(r, CH)], out_hbm.at[cid, pl.ds(r, CH)])

    return _scatter(w, sdst, zrow)


def _e_body(p_ref, x_ref, nt_ref, aw_ref, ac_ref, lw_ref, b_ref, o_ref):
    tot = p_ref[0] + p_ref[1]
    numer = tot[:, :D]
    cols = []
    for h in range(H):
        d = tot[:, D + h:D + h + 1]
        cols.append(numer[:, h * DK:(h + 1) * DK] / (d + 1e-9))
    agg = jnp.concatenate(cols, axis=1)
    y0 = jnp.dot(agg, aw_ref[0], preferred_element_type=jnp.float32)
    y1 = jnp.dot(agg, aw_ref[1], preferred_element_type=jnp.float32)
    nt = nt_ref[...]
    c0 = sum(jnp.where(nt == t, ac_ref[t, 0], 0.0) for t in range(NT))
    c1 = sum(jnp.where(nt == t, ac_ref[t, 1], 0.0) for t in range(NT))
    h_out = c0 * y0 + c1 * y1
    h_out = h_out + jnp.dot(x_ref[...], lw_ref[...],
                            preferred_element_type=jnp.float32)
    h_out = h_out + b_ref[...]
    o_ref[...] = jnp.maximum(h_out, 0.0)


def _finalize(partials, x, node_type, aw, ac, lw, bias):
    full = lambda shp: pl.BlockSpec(shp, lambda i: tuple(0 for _ in shp))
    return pl.pallas_call(
        _e_body,
        grid=(N // BN,),
        in_specs=[
            pl.BlockSpec((NC, BN, WW), lambda i: (0, i, 0)),
            pl.BlockSpec((BN, D), lambda i: (i, 0)),
            pl.BlockSpec((BN, 1), lambda i: (i, 0)),
            full((NB, D, D)), full((NT, NB)), full((D, D)),
            pl.BlockSpec((1, D), lambda i: (0, 0)),
        ],
        out_specs=pl.BlockSpec((BN, D), lambda i: (i, 0)),
        out_shape=jax.ShapeDtypeStruct((N, D), jnp.float32),
    )(partials, x, node_type, aw, ac, lw, bias)


def kernel(x, edge_index, node_type, edge_type, k_w_comp, k_weight,
           q_w_comp, q_weight, v_w_comp, v_weight, a_w_comp, a_weight,
           relation_pri, relation_att, relation_msg, loop_weight, bias):
    src = edge_index[0]
    dst = edge_index[1]
    nt2 = node_type.reshape(N, 1)
    kw = k_weight.reshape(NB, D, D)
    qw = q_weight.reshape(NB, D, D)
    vw = v_weight.reshape(NB, D, D)
    aw = a_weight.reshape(NB, D, D)

    kv, q = _dense_prep(x, nt2, kw, k_w_comp, qw, q_w_comp, vw, v_w_comp,
                        relation_att, relation_msg, relation_pri)
    kv2 = kv.reshape(ET * N, 2 * D)

    pad = EPAD - E
    gidx = jnp.concatenate([edge_type * N + src,
                            jnp.zeros((pad,), jnp.int32)]).reshape(NW, CPW, CH)
    didx = jnp.concatenate([dst, jnp.zeros((pad,), jnp.int32)]
                           ).reshape(NW, CPW, CH)
    sdst = jnp.concatenate([dst, jnp.full((pad,), N, jnp.int32)]
                           ).reshape(NW, CPW, CH)

    kvg, qg = _gather_call(kv2, q, gidx, didx)
    w = _edge_math(kvg, qg)
    partials = _scatter_call(w, sdst, jnp.zeros((CH, WW), jnp.float32))
    return _finalize(partials, x, nt2, aw, a_w_comp, loop_weight,
                     bias.reshape(1, D))


# speedup vs baseline: 34.3446x; 1.0992x over previous
import functools

import jax
import jax.numpy as jnp
from jax import lax
from jax.experimental import pallas as pl
from jax.experimental.pallas import tpu as pltpu
from jax.experimental.pallas import tpu_sc as plsc

N = 10000
E = 320000
D = 128
NT = 3
ET = 5
H = 4
DK = 32
NB = 2
SCALE = float((D / H) ** (-0.5))

NC = 2
NS = 16
NW = NC * NS
CH = 128
CPW = 80
NCHUNK_PAD = CPW * NW
EPAD = NCHUNK_PAD * CH
WW = 144
NP = 10240

BN = 1000
BE = 2048
NBLK_E = EPAD // BE


def _a_body(x_ref, nt_ref, kw_ref, kc_ref, qw_ref, qc_ref, vw_ref, vc_ref,
            aw_att_ref, aw_msg_ref, pri_ref, kv_ref, q_ref):
    x = x_ref[...]
    nt = nt_ref[...]

    def _proj(w_ref, c_ref):
        y0 = jnp.dot(x, w_ref[0], preferred_element_type=jnp.float32)
        y1 = jnp.dot(x, w_ref[1], preferred_element_type=jnp.float32)
        c0 = sum(jnp.where(nt == t, c_ref[t, 0], 0.0) for t in range(NT))
        c1 = sum(jnp.where(nt == t, c_ref[t, 1], 0.0) for t in range(NT))
        return c0 * y0 + c1 * y1

    k = _proj(kw_ref, kc_ref)
    v = _proj(vw_ref, vc_ref)
    q_ref[...] = _proj(qw_ref, qc_ref)

    half = jnp.uint32(0x8000)
    top = jnp.uint32(0xFFFF0000)
    for t in range(ET):
        kcols = []
        vcols = []
        for h in range(H):
            s = pri_ref[t, h] * SCALE
            kcols.append(jnp.dot(k[:, h * DK:(h + 1) * DK], aw_att_ref[t, h],
                                 preferred_element_type=jnp.float32) * s)
            vcols.append(jnp.dot(v[:, h * DK:(h + 1) * DK], aw_msg_ref[t, h],
                                 preferred_element_type=jnp.float32))
        ku = lax.bitcast_convert_type(jnp.concatenate(kcols, axis=1),
                                      jnp.uint32)
        vu = lax.bitcast_convert_type(jnp.concatenate(vcols, axis=1),
                                      jnp.uint32)
        word = ((ku + half) & top) | ((vu + half) >> 16)
        kv_ref[t] = lax.bitcast_convert_type(word, jnp.float32)


def _dense_prep(x, node_type, kw, kc, qw, qc, vw, vc, att_w, msg_w, pri):
    full = lambda shp: pl.BlockSpec(shp, lambda i: tuple(0 for _ in shp))
    return pl.pallas_call(
        _a_body,
        grid=(N // BN,),
        in_specs=[
            pl.BlockSpec((BN, D), lambda i: (i, 0)),
            pl.BlockSpec((BN, 1), lambda i: (i, 0)),
            full((NB, D, D)), full((NT, NB)),
            full((NB, D, D)), full((NT, NB)),
            full((NB, D, D)), full((NT, NB)),
            full((ET, H, DK, DK)), full((ET, H, DK, DK)),
            full((ET, H)),
        ],
        out_specs=[
            pl.BlockSpec((ET, BN, D), lambda i: (0, i, 0)),
            pl.BlockSpec((BN, D), lambda i: (i, 0)),
        ],
        out_shape=[
            jax.ShapeDtypeStruct((ET, N, D), jnp.float32),
            jax.ShapeDtypeStruct((N, D), jnp.float32),
        ],
    )(x, node_type, kw, kc, qw, qc, vw, vc, att_w, msg_w, pri)


def _gather_call(kv, q, gidx, didx):
    mesh = plsc.VectorSubcoreMesh(core_axis_name="c", subcore_axis_name="s")

    @functools.partial(
        pl.kernel, mesh=mesh,
        out_type=[
            jax.ShapeDtypeStruct((EPAD, D), jnp.float32),
            jax.ShapeDtypeStruct((EPAD, D), jnp.float32),
        ],
        scratch_types=[
            pltpu.VMEM((CPW, CH), jnp.int32),
            pltpu.VMEM((CPW, CH), jnp.int32),
            pltpu.VMEM((2, CH, D), jnp.float32),
            pltpu.VMEM((2, CH, D), jnp.float32),
            [pltpu.SemaphoreType.DMA] * 8,
        ],
    )
    def _gather(kv_hbm, q_hbm, gidx_hbm, didx_hbm, okv_hbm, oq_hbm,
                gidx_v, didx_v, kvbuf, qbuf, sems):
        wid = lax.axis_index("c") * NS + lax.axis_index("s")
        sg_kv, sg_q, so_kv, so_q = sems[0:2], sems[2:4], sems[4:6], sems[6:8]
        base = wid * CPW * CH

        pltpu.sync_copy(gidx_hbm.at[wid], gidx_v)
        pltpu.sync_copy(didx_hbm.at[wid], didx_v)

        def start_gather(j, b):
            pltpu.make_async_copy(kv_hbm.at[gidx_v.at[j]], kvbuf.at[b],
                                  sg_kv[b]).start()
            pltpu.make_async_copy(q_hbm.at[didx_v.at[j]], qbuf.at[b],
                                  sg_q[b]).start()

        def wait_gather(j, b):
            pltpu.make_async_copy(kv_hbm.at[gidx_v.at[j]], kvbuf.at[b],
                                  sg_kv[b]).wait()
            pltpu.make_async_copy(q_hbm.at[didx_v.at[j]], qbuf.at[b],
                                  sg_q[b]).wait()

        def start_out(j, b):
            g = base + j * CH
            pltpu.make_async_copy(kvbuf.at[b], okv_hbm.at[pl.ds(g, CH)],
                                  so_kv[b]).start()
            pltpu.make_async_copy(qbuf.at[b], oq_hbm.at[pl.ds(g, CH)],
                                  so_q[b]).start()

        def wait_out(b):
            pltpu.make_async_copy(kvbuf.at[b], okv_hbm.at[pl.ds(0, CH)],
                                  so_kv[b]).wait()
            pltpu.make_async_copy(qbuf.at[b], oq_hbm.at[pl.ds(0, CH)],
                                  so_q[b]).wait()

        start_gather(0, 0)

        @pl.loop(0, CPW // 2)
        def _(jj):
            j0 = jj * 2
            @pl.when(jj > 0)
            def _():
                wait_out(1)
            start_gather(j0 + 1, 1)
            wait_gather(j0, 0)
            start_out(j0, 0)
            wait_out(0)

            @pl.when(jj < CPW // 2 - 1)
            def _():
                start_gather(j0 + 2, 0)
            wait_gather(j0 + 1, 1)
            start_out(j0 + 1, 1)

        wait_out(1)

    return _gather(kv, q, gidx, didx)


def _c_body(kv_ref, q_ref, w_ref):
    wu = lax.bitcast_convert_type(kv_ref[...], jnp.uint32)
    krel = lax.bitcast_convert_type(wu & jnp.uint32(0xFFFF0000), jnp.float32)
    vrel = lax.bitcast_convert_type(wu << 16, jnp.float32)
    prod = krel * q_ref[...]
    chunks = []
    exps = []
    for h in range(H):
        a = jnp.sum(prod[:, h * DK:(h + 1) * DK], axis=1, keepdims=True)
        e = jnp.exp(a)
        exps.append(e)
        chunks.append(vrel[:, h * DK:(h + 1) * DK] * e)
    tail = jnp.concatenate(exps + [jnp.zeros((BE, WW - D - H), jnp.float32)],
                           axis=1)
    w_ref[...] = jnp.concatenate(chunks + [tail], axis=1)


def _edge_math(kvg, qg):
    return pl.pallas_call(
        _c_body,
        grid=(NBLK_E,),
        in_specs=[
            pl.BlockSpec((BE, D), lambda i: (i, 0)),
            pl.BlockSpec((BE, D), lambda i: (i, 0)),
        ],

        out_specs=pl.BlockSpec((BE, WW), lambda i: (i, 0)),
        out_shape=jax.ShapeDtypeStruct((EPAD, WW), jnp.float32),
    )(kvg, qg)


def _scatter_call(w, sdst, zrow):
    mesh = plsc.VectorSubcoreMesh(core_axis_name="c", subcore_axis_name="s")
    rps = NP // NS
    zc = rps // CH

    @functools.partial(
        pl.kernel, mesh=mesh,
        out_type=jax.ShapeDtypeStruct((NC, NP, WW), jnp.float32),
        compiler_params=pltpu.CompilerParams(use_tc_tiling_on_sc=False),
        scratch_types=[
            pltpu.VMEM((2, CH), jnp.int32),
            pltpu.VMEM((2, CH, WW), jnp.float32),
            pltpu.VMEM_SHARED((NP, WW), jnp.float32),
            [pltpu.SemaphoreType.DMA] * 4,
        ],
    )
    def _scatter(w_hbm, sdst_hbm, z_hbm, out_hbm, ibuf, wbuf, acc, sems):
        cid = lax.axis_index("c")
        sid = lax.axis_index("s")
        wid = cid * NS + sid
        base = wid * CPW * CH
        sw, si = sems[0:2], sems[2:4]

        @pl.loop(0, zc)
        def _(z):
            pltpu.sync_copy(z_hbm, acc.at[pl.ds(sid * rps + z * CH, CH)])

        plsc.subcore_barrier()

        def start_ld(j, b):
            pltpu.make_async_copy(w_hbm.at[pl.ds(base + j * CH, CH)],
                                  wbuf.at[b], sw[b]).start()
            pltpu.make_async_copy(sdst_hbm.at[wid, j], ibuf.at[b],
                                  si[b]).start()

        def wait_ld(j, b):
            pltpu.make_async_copy(w_hbm.at[pl.ds(base + j * CH, CH)],
                                  wbuf.at[b], sw[b]).wait()
            pltpu.make_async_copy(sdst_hbm.at[wid, j], ibuf.at[b],
                                  si[b]).wait()

        def scat(j, b):
            pltpu.sync_copy(wbuf.at[b], acc.at[ibuf.at[b]], add=True)

        start_ld(0, 0)

        @pl.loop(0, CPW // 2)
        def _(jj):
            j0 = jj * 2
            start_ld(j0 + 1, 1)
            wait_ld(j0, 0)
            scat(j0, 0)

            @pl.when(jj < CPW // 2 - 1)
            def _():
                start_ld(j0 + 2, 0)
            wait_ld(j0 + 1, 1)
            scat(j0 + 1, 1)

        plsc.subcore_barrier()

        @pl.loop(0, zc)
        def _(z):
            r = sid * rps + z * CH
            pltpu.sync_copy(acc.at[pl.ds(r, CH)], out_hbm.at[cid, pl.ds(r, CH)])

    return _scatter(w, sdst, zrow)


def _e_body(p_ref, x_ref, nt_ref, aw_ref, ac_ref, lw_ref, b_ref, o_ref):
    tot = p_ref[0] + p_ref[1]
    numer = tot[:, :D]
    cols = []
    for h in range(H):
        d = tot[:, D + h:D + h + 1]
        cols.append(numer[:, h * DK:(h + 1) * DK] / (d + 1e-9))
    agg = jnp.concatenate(cols, axis=1)
    y0 = jnp.dot(agg, aw_ref[0], preferred_element_type=jnp.float32)
    y1 = jnp.dot(agg, aw_ref[1], preferred_element_type=jnp.float32)
    nt = nt_ref[...]
    c0 = sum(jnp.where(nt == t, ac_ref[t, 0], 0.0) for t in range(NT))
    c1 = sum(jnp.where(nt == t, ac_ref[t, 1], 0.0) for t in range(NT))
    h_out = c0 * y0 + c1 * y1
    h_out = h_out + jnp.dot(x_ref[...], lw_ref[...],
                            preferred_element_type=jnp.float32)
    h_out = h_out + b_ref[...]
    o_ref[...] = jnp.maximum(h_out, 0.0)


def _finalize(partials, x, node_type, aw, ac, lw, bias):
    full = lambda shp: pl.BlockSpec(shp, lambda i: tuple(0 for _ in shp))
    return pl.pallas_call(
        _e_body,
        grid=(N // BN,),
        in_specs=[
            pl.BlockSpec((NC, BN, WW), lambda i: (0, i, 0)),
            pl.BlockSpec((BN, D), lambda i: (i, 0)),
            pl.BlockSpec((BN, 1), lambda i: (i, 0)),
            full((NB, D, D)), full((NT, NB)), full((D, D)),
            pl.BlockSpec((1, D), lambda i: (0, 0)),
        ],
        out_specs=pl.BlockSpec((BN, D), lambda i: (i, 0)),
        out_shape=jax.ShapeDtypeStruct((N, D), jnp.float32),
    )(partials, x, node_type, aw, ac, lw, bias)


def kernel(x, edge_index, node_type, edge_type, k_w_comp, k_weight,
           q_w_comp, q_weight, v_w_comp, v_weight, a_w_comp, a_weight,
           relation_pri, relation_att, relation_msg, loop_weight, bias):
    src = edge_index[0]
    dst = edge_index[1]
    nt2 = node_type.reshape(N, 1)
    kw = k_weight.reshape(NB, D, D)
    qw = q_weight.reshape(NB, D, D)
    vw = v_weight.reshape(NB, D, D)
    aw = a_weight.reshape(NB, D, D)

    kv, q = _dense_prep(x, nt2, kw, k_w_comp, qw, q_w_comp, vw, v_w_comp,
                        relation_att, relation_msg, relation_pri)
    kv2 = kv.reshape(ET * N, D)

    pad = EPAD - E
    gidx = jnp.concatenate([edge_type * N + src,
                            jnp.zeros((pad,), jnp.int32)]).reshape(NW, CPW, CH)
    didx = jnp.concatenate([dst, jnp.zeros((pad,), jnp.int32)]
                           ).reshape(NW, CPW, CH)
    sdst = jnp.concatenate([dst, jnp.full((pad,), N, jnp.int32)]
                           ).reshape(NW, CPW, CH)

    kvg, qg = _gather_call(kv2, q, gidx, didx)
    w = _edge_math(kvg, qg)
    partials = _scatter_call(w, sdst, jnp.zeros((CH, WW), jnp.float32))
    return _finalize(partials, x, nt2, aw, a_w_comp, loop_weight,
                     bias.reshape(1, D))
